# baseline (speedup 1.0000x reference)
import functools

import jax
import jax.numpy as jnp
from jax import lax
from jax.experimental import pallas as pl
from jax.experimental.pallas import tpu as pltpu
from jax.experimental.pallas import tpu_sc as plsc

F = 26
V = 100000
D = 32
B = 16384
N = B * F
NC, NS, L = 2, 16, 16
NW = NC * NS
N_PER_W = N // NW
B_PER_W = N_PER_W // F
UNROLL = 8
VEC_ITERS = N_PER_W // (L * UNROLL)
NB = 8
CHUNK = NB * F
N_CHUNKS = N_PER_W // CHUNK
NBUF = 2


def _sc_gather(x_flat, table128):
  mesh = plsc.VectorSubcoreMesh(core_axis_name="c", subcore_axis_name="s")

  @functools.partial(
      pl.kernel,
      out_type=jax.ShapeDtypeStruct((B, F, D), jnp.float32),
      mesh=mesh,
      scratch_types=[
          pltpu.VMEM((N_PER_W,), jnp.int32),
          pltpu.VMEM((CHUNK,), jnp.int32),
          pltpu.VMEM((CHUNK,), jnp.int32),
          pltpu.VMEM((NBUF, CHUNK, 128), jnp.float32),
          pltpu.VMEM((NBUF, CHUNK, D), jnp.float32),
      ] + [pltpu.SemaphoreType.DMA] * NBUF
        + [pltpu.SemaphoreType.DMA] * NBUF,
      compiler_params=pltpu.CompilerParams(use_tc_tiling_on_sc=True),
  )
  def k(x_hbm, tab_hbm, out_hbm, idx_v, gidx0, gidx1, lines, rows, *sems):
    gidx = [gidx0, gidx1]
    gsems = sems[:NBUF]
    wsems = sems[NBUF:]
    wid = lax.axis_index("s") * NC + lax.axis_index("c")
    base = wid * N_PER_W
    b_base = wid * B_PER_W

    pltpu.sync_copy(x_hbm.at[pl.ds(base, N_PER_W)], idx_v)

    lanes = lax.iota(jnp.int32, L)

    def vbody(i, carry):
      for j in range(UNROLL):
        off = (i * UNROLL + j) * L
        fld = lax.rem(off + lanes, F)
        idx_v[pl.ds(off, L)] = idx_v[pl.ds(off, L)] + fld * V
      return carry

    lax.fori_loop(0, VEC_ITERS, vbody, 0)

    def gather(c, b):
      def lbody(vi, carry):
        idx_vec = idx_v[pl.ds(c * CHUNK + vi * L, L)]
        gidx[b][pl.ds(vi * L, L)] = lax.shift_right_logical(idx_vec, 2)
        return carry

      lax.fori_loop(0, CHUNK // L, lbody, 0)
      pltpu.async_copy(
          tab_hbm.at[gidx[b]], lines.at[b], gsems[b])

    def gwait(b):
      pltpu.make_async_copy(
          tab_hbm.at[pl.ds(0, CHUNK)], lines.at[b], gsems[b]).wait()

    def extract(c, b):
      def ebody(vi, carry):
        flat = idx_v[pl.ds(c * CHUNK + vi * L, L)]
        ovec = lax.shift_left(lax.bitwise_and(flat, 3), 5)
        r0 = vi * L
        for j in range(L):
          r = r0 + j
          o = ovec[j]
          rows[b, r, pl.ds(0, L)] = lines[b, r, pl.ds(o, L)]
          rows[b, r, pl.ds(L, L)] = lines[b, r, pl.ds(o + L, L)]
        return carry

      lax.fori_loop(0, CHUNK // L, ebody, 0)

    def writeout(c, b):
      for bb in range(NB):
        pltpu.async_copy(
            rows.at[b].at[pl.ds(bb * F, F)],
            out_hbm.at[b_base + c * NB + bb], wsems[b])

    def wwait(b):
      for bb in range(NB):
        pltpu.make_async_copy(
            rows.at[b].at[pl.ds(bb * F, F)],
            out_hbm.at[b_base], wsems[b]).wait()

    gather(0, 0)

    def pbody(g, carry):
      for b in range(NBUF):
        c = g * NBUF + b
        nxt = lax.rem(c + 1, N_CHUNKS)
        gather(nxt, 1 - b)
        gwait(b)
        @pl.when(c >= NBUF)
        def _():
          wwait(b)
        extract(c, b)
        writeout(c, b)
      return carry

    lax.fori_loop(0, N_CHUNKS // NBUF, pbody, 0)
    gwait(0)
    for b in range(NBUF):
      wwait(b)

  return k(x_flat, table128)


def kernel(x_cat, tables):
  return _sc_gather(x_cat.reshape(N), tables.reshape(F * V // 4, 128))


# speedup vs baseline: 2.2929x; 2.2929x over previous
import functools

import jax
import jax.numpy as jnp
from jax import lax
from jax.experimental import pallas as pl
from jax.experimental.pallas import tpu as pltpu
from jax.experimental.pallas import tpu_sc as plsc

F = 26
V = 100000
D = 32
B = 16384
N = B * F
NC, NS, L = 2, 16, 16
NW = NC * NS
N_PER_W = N // NW
B_PER_W = N_PER_W // F
UNROLL = 8
VEC_ITERS = N_PER_W // (L * UNROLL)
NB = 8
CHUNK = NB * F
N_CHUNKS = N_PER_W // CHUNK
NBUF = 2


def _sc_gather(x_flat, table_flat):
  mesh = plsc.VectorSubcoreMesh(core_axis_name="c", subcore_axis_name="s")

  @functools.partial(
      pl.kernel,
      out_type=jax.ShapeDtypeStruct((B, F, D), jnp.float32),
      mesh=mesh,
      scratch_types=[
          pltpu.VMEM((N_PER_W,), jnp.int32),
          pltpu.VMEM((NBUF, CHUNK, D), jnp.float32),
      ] + [pltpu.SemaphoreType.DMA] * NBUF
        + [pltpu.SemaphoreType.DMA] * NBUF,
      compiler_params=pltpu.CompilerParams(use_tc_tiling_on_sc=True),
  )
  def k(x_hbm, tab_hbm, out_hbm, idx_v, rows, *sems):
    gsems = sems[:NBUF]
    wsems = sems[NBUF:]
    wid = lax.axis_index("s") * NC + lax.axis_index("c")
    base = wid * N_PER_W
    b_base = wid * B_PER_W

    pltpu.sync_copy(x_hbm.at[pl.ds(base, N_PER_W)], idx_v)

    lanes = lax.iota(jnp.int32, L)

    def vbody(i, carry):
      for j in range(UNROLL):
        off = (i * UNROLL + j) * L
        fld = lax.rem(off + lanes, F)
        idx_v[pl.ds(off, L)] = idx_v[pl.ds(off, L)] + fld * V
      return carry

    lax.fori_loop(0, VEC_ITERS, vbody, 0)

    def gather(c, b):
      def gbody(vi, carry):
        ivec = idx_v[pl.ds(c * CHUNK + vi * L, L)]
        r0 = vi * L
        for j in range(L):
          pltpu.async_copy(
              tab_hbm.at[ivec[j]], rows.at[b].at[r0 + j], gsems[b])
        return carry

      lax.fori_loop(0, CHUNK // L, gbody, 0)

    def gwait(b):
      pltpu.make_async_copy(
          tab_hbm.at[pl.ds(0, CHUNK)], rows.at[b], gsems[b]).wait()

    def writeout(c, b):
      for bb in range(NB):
        pltpu.async_copy(
            rows.at[b].at[pl.ds(bb * F, F)],
            out_hbm.at[b_base + c * NB + bb], wsems[b])

    def wwait(b):
      for bb in range(NB):
        pltpu.make_async_copy(
            rows.at[b].at[pl.ds(bb * F, F)],
            out_hbm.at[b_base], wsems[b]).wait()

    gather(0, 0)

    def pbody(g, carry):
      for b in range(NBUF):
        c = g * NBUF + b
        nxt = lax.rem(c + 1, N_CHUNKS)
        gwait(b)

        @pl.when(c >= 1)
        def _():
          wwait(1 - b)
        writeout(c, b)
        gather(nxt, 1 - b)
      return carry

    lax.fori_loop(0, N_CHUNKS // NBUF, pbody, 0)
    gwait(0)
    wwait(1)

  return k(x_flat, table_flat)


def kernel(x_cat, tables):
  return _sc_gather(x_cat.reshape(N), tables.reshape(F * V, D))


# speedup vs baseline: 2.3154x; 1.0098x over previous
import functools

import jax
import jax.numpy as jnp
from jax import lax
from jax.experimental import pallas as pl
from jax.experimental.pallas import tpu as pltpu
from jax.experimental.pallas import tpu_sc as plsc

F = 26
V = 100000
D = 32
B = 16384
N = B * F
NC, NS, L = 2, 16, 16
NW = NC * NS
N_PER_W = N // NW
B_PER_W = N_PER_W // F
UNROLL = 8
VEC_ITERS = N_PER_W // (L * UNROLL)
NB = 16
CHUNK = NB * F
N_CHUNKS = N_PER_W // CHUNK
NBUF = 2


def _sc_gather(x_flat, table_flat):
  mesh = plsc.VectorSubcoreMesh(core_axis_name="c", subcore_axis_name="s")

  @functools.partial(
      pl.kernel,
      out_type=jax.ShapeDtypeStruct((B, F, D), jnp.float32),
      mesh=mesh,
      scratch_types=[
          pltpu.VMEM((N_PER_W,), jnp.int32),
          pltpu.VMEM((NBUF, CHUNK, D), jnp.float32),
      ] + [pltpu.SemaphoreType.DMA] * NBUF
        + [pltpu.SemaphoreType.DMA] * NBUF,
      compiler_params=pltpu.CompilerParams(use_tc_tiling_on_sc=True),
  )
  def k(x_hbm, tab_hbm, out_hbm, idx_v, rows, *sems):
    gsems = sems[:NBUF]
    wsems = sems[NBUF:]
    wid = lax.axis_index("s") * NC + lax.axis_index("c")
    base = wid * N_PER_W
    b_base = wid * B_PER_W

    pltpu.sync_copy(x_hbm.at[pl.ds(base, N_PER_W)], idx_v)

    lanes = lax.iota(jnp.int32, L)

    def vbody(i, carry):
      for j in range(UNROLL):
        off = (i * UNROLL + j) * L
        fld = lax.rem(off + lanes, F)
        idx_v[pl.ds(off, L)] = idx_v[pl.ds(off, L)] + fld * V
      return carry

    lax.fori_loop(0, VEC_ITERS, vbody, 0)

    def gather(c, b):
      def gbody(vi, carry):
        ivec = idx_v[pl.ds(c * CHUNK + vi * L, L)]
        r0 = vi * L
        for j in range(L):
          pltpu.async_copy(
              tab_hbm.at[ivec[j]], rows.at[b].at[r0 + j], gsems[b])
        return carry

      lax.fori_loop(0, CHUNK // L, gbody, 0)

    def gwait(b):
      pltpu.make_async_copy(
          tab_hbm.at[pl.ds(0, CHUNK)], rows.at[b], gsems[b]).wait()

    def writeout(c, b):
      for bb in range(NB):
        pltpu.async_copy(
            rows.at[b].at[pl.ds(bb * F, F)],
            out_hbm.at[b_base + c * NB + bb], wsems[b])

    def wwait(b):
      for bb in range(NB):
        pltpu.make_async_copy(
            rows.at[b].at[pl.ds(bb * F, F)],
            out_hbm.at[b_base], wsems[b]).wait()

    gather(0, 0)

    def pbody(g, carry):
      for b in range(NBUF):
        c = g * NBUF + b
        nxt = lax.rem(c + 1, N_CHUNKS)
        gwait(b)

        @pl.when(c >= 1)
        def _():
          wwait(1 - b)
        writeout(c, b)
        gather(nxt, 1 - b)
      return carry

    lax.fori_loop(0, N_CHUNKS // NBUF, pbody, 0)
    gwait(0)
    wwait(1)

  return k(x_flat, table_flat)


def kernel(x_cat, tables):
  return _sc_gather(x_cat.reshape(N), tables.reshape(F * V, D))
